# baseline (speedup 1.0000x reference)
import functools

import jax
import jax.numpy as jnp
from jax import lax
from jax.experimental import pallas as pl
from jax.experimental.pallas import tpu as pltpu
from jax.experimental.pallas import tpu_sc as plsc

DIM = 300
SEQ = 200
TOK = 180
BATCH = 1024
GRAN = 8
GPT = 38
SROWS = TOK * GPT
DROWS = TOK * DIM // GRAN
OROWS = SEQ * DIM // GRAN
ZROWS = OROWS - DROWS
TOK_PAD = 192
CHUNKS = 19


@functools.lru_cache(maxsize=1)
def _make_sc_gather():
    info = plsc.get_sparse_core_info()
    nw = info.num_cores * info.num_subcores
    bpw = BATCH // nw
    mesh = plsc.VectorSubcoreMesh(core_axis_name="c", subcore_axis_name="s")

    @functools.partial(
        pl.kernel,
        mesh=mesh,
        compiler_params=pltpu.CompilerParams(
            use_tc_tiling_on_sc=False, needs_layout_passes=False),
        out_type=jax.ShapeDtypeStruct((BATCH, OROWS, GRAN), jnp.float32),
        scratch_types=[
            pltpu.VMEM((SROWS,), jnp.int32),
            pltpu.VMEM((SROWS,), jnp.int32),
            pltpu.VMEM((TOK_PAD,), jnp.int32),
            pltpu.VMEM((TOK_PAD,), jnp.int32),
            pltpu.VMEM((SROWS, GRAN), jnp.float32),
            pltpu.VMEM((SROWS, GRAN), jnp.float32),
            pltpu.VMEM((ZROWS, GRAN), jnp.float32),
            pltpu.SemaphoreType.DMA,
            pltpu.SemaphoreType.DMA,
            pltpu.SemaphoreType.DMA,
            pltpu.SemaphoreType.DMA,
            pltpu.SemaphoreType.DMA,
            pltpu.SemaphoreType.DMA,
        ],
    )
    def gather_kernel(gl_hbm, ph_hbm, zeros_hbm, tableg_hbm, out_hbm,
                      gl0, gl1, ph0, ph1, st0, st1, zbuf,
                      g0, g1, s0, s1, z0, z1):
        wid = lax.axis_index("s") * info.num_cores + lax.axis_index("c")
        b0 = wid * bpw
        pltpu.sync_copy(zeros_hbm, zbuf)

        viota = lax.iota(jnp.int32, 16)
        row_p0 = viota >> 3
        col_p0 = viota & 7
        row_p4 = (viota + 4) >> 3
        col_p4 = (viota + 4) & 7

        def extract(st, ph):
            def token_block(g, pv, u):
                t = 16 * g + u
                phi = pv[u]
                is4 = phi != 0
                srow = jnp.where(is4, row_p4, row_p0)
                scol = jnp.where(is4, col_p4, col_p0)
                psi = 4 * (u & 1)
                drow = row_p4 if (u & 1) else row_p0
                dcol = col_p4 if (u & 1) else col_p0
                sbase = GPT * t
                dbase = (DIM * t - psi) >> 3
                for k in range(CHUNKS):
                    x = plsc.load_gather(st, [srow + (sbase + 2 * k), scol])
                    plsc.store_scatter(st, [drow + (dbase + 2 * k), dcol], x)

            def grp(g, c):
                base = pl.multiple_of(16 * g, 16)
                pv = ph[pl.ds(base, 16)]
                for u in range(16):
                    token_block(g, pv, u)
                return c

            lax.fori_loop(0, TOK // 16, grp, 0)
            pv = ph[pl.ds(16 * (TOK // 16), 16)]
            for u in range(TOK - 16 * (TOK // 16)):
                token_block(TOK // 16, pv, u)

        def load_and_fire(j, gl, ph, st, gsem):
            pltpu.sync_copy(gl_hbm.at[b0 + j], gl)
            pltpu.sync_copy(ph_hbm.at[b0 + j], ph)
            return pltpu.async_copy(tableg_hbm.at[gl], st, gsem)

        def store_sentence(j, st, ssem, zsem):
            a = pltpu.async_copy(st.at[pl.ds(0, DROWS)],
                                 out_hbm.at[b0 + j, pl.ds(0, DROWS)], ssem)
            z = pltpu.async_copy(zbuf,
                                 out_hbm.at[b0 + j, pl.ds(DROWS, ZROWS)], zsem)
            return a, z

        def body(i, carry):
            j0 = 2 * i
            j1 = 2 * i + 1
            ga = load_and_fire(j0, gl0, ph0, st0, g0)
            gb = load_and_fire(j1, gl1, ph1, st1, g1)
            ga.wait()
            extract(st0, ph0)
            sa, za = store_sentence(j0, st0, s0, z0)
            gb.wait()
            extract(st1, ph1)
            sb, zb = store_sentence(j1, st1, s1, z1)
            sa.wait()
            za.wait()
            sb.wait()
            zb.wait()
            return carry

        lax.fori_loop(0, bpw // 2, body, 0)

    return gather_kernel


def kernel(indices, table):
    flat = indices.reshape(-1)
    start = (flat * 75) >> 1
    gl = (start[:, None] + jnp.arange(GPT, dtype=jnp.int32)).reshape(BATCH, SROWS)
    ph = ((flat & 1) * 4).reshape(BATCH, TOK)
    ph = jnp.pad(ph, ((0, 0), (0, TOK_PAD - TOK)))
    zeros = jnp.zeros((ZROWS, GRAN), jnp.float32)
    tg = table.reshape(-1, GRAN)
    out = _make_sc_gather()(gl, ph, zeros, tg)
    return out.reshape(BATCH, SEQ, DIM)


# speedup vs baseline: 1.0938x; 1.0938x over previous
import functools

import jax
import jax.numpy as jnp
from jax import lax
from jax.experimental import pallas as pl
from jax.experimental.pallas import tpu as pltpu
from jax.experimental.pallas import tpu_sc as plsc

DIM = 300
SEQ = 200
TOK = 180
BATCH = 1024
GRAN = 8
GPT = 38
SROWS = TOK * GPT
DWORDS = TOK * DIM
OWORDS = SEQ * DIM
ZWORDS = OWORDS - DWORDS
TOK_PAD = 192
CHUNKS = 19


@functools.lru_cache(maxsize=1)
def _make_sc_gather():
    info = plsc.get_sparse_core_info()
    nw = info.num_cores * info.num_subcores
    bpw = BATCH // nw
    mesh = plsc.VectorSubcoreMesh(core_axis_name="c", subcore_axis_name="s")

    @functools.partial(
        pl.kernel,
        mesh=mesh,
        compiler_params=pltpu.CompilerParams(
            use_tc_tiling_on_sc=False, needs_layout_passes=False),
        out_type=jax.ShapeDtypeStruct((BATCH, OWORDS), jnp.float32),
        scratch_types=[
            pltpu.VMEM((SROWS,), jnp.int32),
            pltpu.VMEM((TOK_PAD,), jnp.int32),
            pltpu.VMEM((SROWS, GRAN), jnp.float32),
            pltpu.VMEM((DWORDS + 16,), jnp.float32),
            pltpu.VMEM((ZWORDS,), jnp.float32),
            pltpu.SemaphoreType.DMA,
            pltpu.SemaphoreType.DMA,
            pltpu.SemaphoreType.DMA,
        ],
    )
    def gather_kernel(gl_hbm, ph_hbm, zeros_hbm, tableg_hbm, out_hbm,
                      glv, phv, stg, pkd, zbuf, gsem, ssem, zsem):
        wid = lax.axis_index("s") * info.num_cores + lax.axis_index("c")
        b0 = wid * bpw
        pltpu.sync_copy(zeros_hbm, zbuf)

        viota = lax.iota(jnp.int32, 16)
        row_p0 = viota >> 3
        col_p0 = viota & 7
        row_p4 = (viota + 4) >> 3
        col_p4 = (viota + 4) & 7

        def extract():
            def token_block(g, pv, u):
                t = 16 * g + u
                phi = pv[u]
                is4 = phi != 0
                srow = jnp.where(is4, row_p4, row_p0)
                scol = jnp.where(is4, col_p4, col_p0)
                sbase = GPT * t
                dbase = DIM * t
                for k in range(CHUNKS):
                    x = plsc.load_gather(stg, [srow + (sbase + 2 * k), scol])
                    plsc.store_scatter(pkd, [viota + (dbase + 16 * k)], x)

            def grp(g, c):
                base = pl.multiple_of(16 * g, 16)
                pv = phv[pl.ds(base, 16)]
                for u in range(16):
                    token_block(g, pv, u)
                return c

            lax.fori_loop(0, TOK // 16, grp, 0)
            pv = phv[pl.ds(16 * (TOK // 16), 16)]
            for u in range(TOK - 16 * (TOK // 16)):
                token_block(TOK // 16, pv, u)

        def drain(sem, dst):
            pltpu.make_async_copy(pkd.at[pl.ds(0, dst.shape[0])], dst, sem).wait()

        def body(j, carry):
            pltpu.sync_copy(gl_hbm.at[b0 + j], glv)
            pltpu.sync_copy(ph_hbm.at[b0 + j], phv)
            g = pltpu.async_copy(tableg_hbm.at[glv], stg, gsem)

            @pl.when(j >= 1)
            def _():
                drain(ssem, out_hbm.at[b0, pl.ds(0, DWORDS)])
                drain(zsem, out_hbm.at[b0, pl.ds(DWORDS, ZWORDS)])

            g.wait()
            extract()
            pltpu.async_copy(pkd.at[pl.ds(0, DWORDS)],
                             out_hbm.at[b0 + j, pl.ds(0, DWORDS)], ssem)
            pltpu.async_copy(zbuf,
                             out_hbm.at[b0 + j, pl.ds(DWORDS, ZWORDS)], zsem)
            return carry

        lax.fori_loop(0, bpw, body, 0)
        drain(ssem, out_hbm.at[b0, pl.ds(0, DWORDS)])
        drain(zsem, out_hbm.at[b0, pl.ds(DWORDS, ZWORDS)])

    return gather_kernel


def kernel(indices, table):
    flat = indices.reshape(-1)
    start = (flat * 75) >> 1
    gl = (start[:, None] + jnp.arange(GPT, dtype=jnp.int32)).reshape(BATCH, SROWS)
    ph = ((flat & 1) * 4).reshape(BATCH, TOK)
    ph = jnp.pad(ph, ((0, 0), (0, TOK_PAD - TOK)))
    zeros = jnp.zeros((ZWORDS,), jnp.float32)
    tg = table.reshape(-1, GRAN)
    out = _make_sc_gather()(gl, ph, zeros, tg)
    return out.reshape(BATCH, SEQ, DIM)
